# Initial kernel scaffold; baseline (speedup 1.0000x reference)
import jax
import jax.numpy as jnp
from jax.experimental import pallas as pl


def kernel(x, edge_index, W1_l, W1_r, b1, g1, be1, W2_l, W2_r, b2, g2, be2):
    raise NotImplementedError("write your pallas kernel here")


# speedup vs baseline: 5.7661x; 5.7661x over previous
import jax
import jax.numpy as jnp
from jax import lax
from jax.experimental import pallas as pl
from jax.experimental.pallas import tpu as pltpu
from jax.experimental.pallas import tpu_sc as plsc

N = 10000
E = 320000
D1 = 128
D2 = 64
NC = 2
NS = 16
EPT = E // NS
K = 80
CPB = 25
BPT = EPT // (K * CPB)
RPT = 632
NP = NS * RPT
CW = 16

_mesh = plsc.VectorSubcoreMesh(core_axis_name="c", subcore_axis_name="s")


def _make_sc_agg(DH, with_counts):
    out_type = [jax.ShapeDtypeStruct((NC, NP, DH), jnp.float32)]
    scratch = [
        pltpu.VMEM((CPB, K), jnp.int32),
        pltpu.VMEM((CPB, K), jnp.int32),
        pltpu.VMEM((K, DH), jnp.float32),
        pltpu.VMEM_SHARED((NP, DH), jnp.float32),
        pltpu.SemaphoreType.DMA,
    ]
    if with_counts:
        out_type.append(jax.ShapeDtypeStruct((NP, CW), jnp.float32))
        scratch += [
            pltpu.VMEM((K, CW), jnp.float32),
            pltpu.VMEM_SHARED((NP, CW), jnp.float32),
        ]

    def body(src_hbm, dst_hbm, feat_hbm, zf_hbm, *rest):
        if with_counts:
            (zc_hbm, sum_out, cnt_out,
             src_v, dst_v, rows_v, sum_sh, sem, ones_v, cnt_sh) = rest
        else:
            (sum_out, src_v, dst_v, rows_v, sum_sh, sem) = rest
        cid = lax.axis_index("c")
        sid = lax.axis_index("s")

        pltpu.sync_copy(zf_hbm.at[pl.ds(sid * RPT, RPT)],
                        sum_sh.at[pl.ds(sid * RPT, RPT)])
        if with_counts:
            pltpu.sync_copy(zc_hbm.at[pl.ds(sid * RPT, RPT)],
                            cnt_sh.at[pl.ds(sid * RPT, RPT)])
            ones = jnp.ones((CW,), jnp.float32)

            def fill(i, carry):
                ones_v[i] = ones
                return carry

            lax.fori_loop(0, K, fill, 0)
        plsc.subcore_barrier()

        def block(b, carry):
            pltpu.sync_copy(src_hbm.at[sid, b], src_v)
            pltpu.sync_copy(dst_hbm.at[sid, b], dst_v)

            def step(j, carry2):
                pltpu.async_copy(feat_hbm.at[cid].at[src_v.at[j]],
                                 rows_v, sem).wait()
                pltpu.sync_copy(rows_v, sum_sh.at[dst_v.at[j]], add=True)
                if with_counts:
                    @pl.when(cid == 0)
                    def _():
                        pltpu.sync_copy(ones_v, cnt_sh.at[dst_v.at[j]],
                                        add=True)
                return carry2

            lax.fori_loop(0, CPB, step, 0)
            return carry

        lax.fori_loop(0, BPT, block, 0)
        plsc.subcore_barrier()

        pltpu.sync_copy(sum_sh.at[pl.ds(sid * RPT, RPT)],
                        sum_out.at[cid, pl.ds(sid * RPT, RPT)])
        if with_counts:
            @pl.when(cid == 0)
            def _():
                pltpu.sync_copy(cnt_sh.at[pl.ds(sid * RPT, RPT)],
                                cnt_out.at[pl.ds(sid * RPT, RPT)])

    return pl.kernel(body, out_type=out_type, mesh=_mesh,
                     scratch_types=scratch,
                     compiler_params=pltpu.CompilerParams(
                         use_tc_tiling_on_sc=False))


_sc_agg_l1 = _make_sc_agg(D1 // NC, with_counts=True)
_sc_agg_l2 = _make_sc_agg(D2 // NC, with_counts=False)


def _tc1_body(x, s, c, w1l, w1r, b1, g1, be1, w2l, h1_out, y2_out):
    cnt = jnp.maximum(c[:N, 0:1], 1.0)
    h = (jnp.dot(s[0, :N] / cnt, w1l[0:D1 // 2, :],
                 preferred_element_type=jnp.float32)
         + jnp.dot(s[1, :N] / cnt, w1l[D1 // 2:, :],
                   preferred_element_type=jnp.float32)
         + jnp.dot(x[...], w1r[...], preferred_element_type=jnp.float32)
         + b1[...])
    mean = jnp.mean(h, axis=0, keepdims=True)
    var = jnp.mean((h - mean) ** 2, axis=0, keepdims=True)
    h = g1[...] * (h - mean) * lax.rsqrt(var + 1e-5) + be1[...]
    h = jnp.maximum(h, 0.0)
    h1_out[...] = h
    y2_out[...] = jnp.dot(h, w2l[...], preferred_element_type=jnp.float32)


def _tc2_body(s, c, h1, w2r, b2, g2, be2, out):
    cnt = jnp.maximum(c[:N, 0:1], 1.0)
    aggp = jnp.concatenate([s[0, :N], s[1, :N]], axis=1) / cnt
    h = (aggp
         + jnp.dot(h1[...], w2r[...], preferred_element_type=jnp.float32)
         + b2[...])
    mean = jnp.mean(h, axis=0, keepdims=True)
    var = jnp.mean((h - mean) ** 2, axis=0, keepdims=True)
    h = g2[...] * (h - mean) * lax.rsqrt(var + 1e-5) + be2[...]
    out[...] = jnp.maximum(h, 0.0)


_tc1 = pl.pallas_call(
    _tc1_body,
    out_shape=[jax.ShapeDtypeStruct((N, D1), jnp.float32),
               jax.ShapeDtypeStruct((N, D2), jnp.float32)],
)

_tc2 = pl.pallas_call(
    _tc2_body,
    out_shape=jax.ShapeDtypeStruct((N, D2), jnp.float32),
)


def kernel(x, edge_index, W1_l, W1_r, b1, g1, be1, W2_l, W2_r, b2, g2, be2):
    src = edge_index[0].astype(jnp.int32).reshape(NS, BPT, CPB, K)
    dst = edge_index[1].astype(jnp.int32).reshape(NS, BPT, CPB, K)
    zf1 = jnp.zeros((NP, D1 // NC), jnp.float32)
    zc = jnp.zeros((NP, CW), jnp.float32)
    zf2 = jnp.zeros((NP, D2 // NC), jnp.float32)

    xh = jnp.stack([x[:, :D1 // 2], x[:, D1 // 2:]])
    sum1, cnt = _sc_agg_l1(src, dst, xh, zf1, zc)
    h1, y2 = _tc1(x, sum1, cnt,
                  W1_l, W1_r, b1.reshape(1, D1), g1.reshape(1, D1),
                  be1.reshape(1, D1), W2_l)
    yh = jnp.stack([y2[:, :D2 // 2], y2[:, D2 // 2:]])
    (sum2,) = _sc_agg_l2(src, dst, yh, zf2)
    out = _tc2(sum2, cnt, h1, W2_r, b2.reshape(1, D2),
               g2.reshape(1, D2), be2.reshape(1, D2))
    return out


# speedup vs baseline: 8.5695x; 1.4862x over previous
import jax
import jax.numpy as jnp
from jax import lax
from jax.experimental import pallas as pl
from jax.experimental.pallas import tpu as pltpu
from jax.experimental.pallas import tpu_sc as plsc

N = 10000
E = 320000
D1 = 128
D2 = 64
NC = 2
NS = 16
K = 80
BC = 12
NBLK = 21
CT = NBLK * BC
EP = NS * CT * K
NSLOT = 6
PRE = 3
RPT = 632
NP = NS * RPT
CW = 16

_mesh = plsc.VectorSubcoreMesh(core_axis_name="c", subcore_axis_name="s")


def _make_sc_agg(DH, with_counts):
    out_type = [jax.ShapeDtypeStruct((NC, NP, DH), jnp.float32)]
    scratch = [
        pltpu.VMEM((BC, K), jnp.int32),
        pltpu.VMEM((BC, K), jnp.int32),
        pltpu.VMEM((NSLOT, K, DH), jnp.float32),
        pltpu.VMEM_SHARED((NP, DH), jnp.float32),
        pltpu.SemaphoreType.DMA((NSLOT,)),
        pltpu.SemaphoreType.DMA((NSLOT,)),
    ]
    if with_counts:
        out_type.append(jax.ShapeDtypeStruct((NP, CW), jnp.float32))
        scratch += [
            pltpu.VMEM((K, CW), jnp.float32),
            pltpu.VMEM_SHARED((NP, CW), jnp.float32),
            pltpu.SemaphoreType.DMA((NSLOT,)),
        ]

    def body(src_hbm, dst_hbm, feat_hbm, zf_hbm, *rest):
        if with_counts:
            (zc_hbm, sum_out, cnt_out, src_v, dst_v, rows_v, sum_sh,
             sem_g, sem_s, ones_v, cnt_sh, sem_o) = rest
        else:
            (sum_out, src_v, dst_v, rows_v, sum_sh, sem_g, sem_s) = rest
        cid = lax.axis_index("c")
        sid = lax.axis_index("s")

        pltpu.sync_copy(zf_hbm.at[pl.ds(sid * RPT, RPT)],
                        sum_sh.at[pl.ds(sid * RPT, RPT)])
        if with_counts:
            pltpu.sync_copy(zc_hbm.at[pl.ds(sid * RPT, RPT)],
                            cnt_sh.at[pl.ds(sid * RPT, RPT)])
            ones = jnp.ones((CW,), jnp.float32)

            def fill(i, carry):
                ones_v[i] = ones
                return carry

            lax.fori_loop(0, K, fill, 0)
        plsc.subcore_barrier()

        def start_gather(i, s):
            pltpu.async_copy(feat_hbm.at[cid].at[src_v.at[i]],
                             rows_v.at[s], sem_g.at[s])

        def wait_gather(s):
            pltpu.make_async_copy(feat_hbm.at[cid].at[src_v.at[0]],
                                  rows_v.at[s], sem_g.at[s]).wait()

        def start_scatter(i, s):
            pltpu.async_copy(rows_v.at[s], sum_sh.at[dst_v.at[i]],
                             sem_s.at[s], add=True)
            if with_counts:
                @pl.when(cid == 0)
                def _():
                    pltpu.async_copy(ones_v, cnt_sh.at[dst_v.at[i]],
                                     sem_o.at[s], add=True)

        def wait_scatter(s):
            pltpu.make_async_copy(rows_v.at[s], sum_sh.at[dst_v.at[0]],
                                  sem_s.at[s]).wait()
            if with_counts:
                @pl.when(cid == 0)
                def _():
                    pltpu.make_async_copy(ones_v, cnt_sh.at[dst_v.at[0]],
                                          sem_o.at[s]).wait()

        def block(b, carry):
            pltpu.sync_copy(src_hbm.at[sid, b], src_v)
            pltpu.sync_copy(dst_hbm.at[sid, b], dst_v)
            for i in range(PRE):
                start_gather(i, i)
            for i in range(BC):
                s = i % NSLOT
                if i + PRE < BC:
                    t = (i + PRE) % NSLOT
                    if i + PRE >= NSLOT:
                        wait_scatter(t)
                    start_gather(i + PRE, t)
                wait_gather(s)
                start_scatter(i, s)
            for s in range(NSLOT):
                wait_scatter(s)
            return carry

        lax.fori_loop(0, NBLK, block, 0)
        plsc.subcore_barrier()

        pltpu.sync_copy(sum_sh.at[pl.ds(sid * RPT, RPT)],
                        sum_out.at[cid, pl.ds(sid * RPT, RPT)])
        if with_counts:
            @pl.when(cid == 0)
            def _():
                pltpu.sync_copy(cnt_sh.at[pl.ds(sid * RPT, RPT)],
                                cnt_out.at[pl.ds(sid * RPT, RPT)])

    return pl.kernel(body, out_type=out_type, mesh=_mesh,
                     scratch_types=scratch,
                     compiler_params=pltpu.CompilerParams(
                         use_tc_tiling_on_sc=False))


_sc_agg_l1 = _make_sc_agg(D1 // NC, with_counts=True)
_sc_agg_l2 = _make_sc_agg(D2 // NC, with_counts=False)


def _tc1_body(x, s, c, w1l, w1r, b1, g1, be1, w2l, h1_out, y2_out):
    cnt = jnp.maximum(c[:N, 0:1], 1.0)
    h = (jnp.dot(s[0, :N] / cnt, w1l[0:D1 // 2, :],
                 preferred_element_type=jnp.float32)
         + jnp.dot(s[1, :N] / cnt, w1l[D1 // 2:, :],
                   preferred_element_type=jnp.float32)
         + jnp.dot(x[...], w1r[...], preferred_element_type=jnp.float32)
         + b1[...])
    mean = jnp.mean(h, axis=0, keepdims=True)
    var = jnp.mean((h - mean) ** 2, axis=0, keepdims=True)
    h = g1[...] * (h - mean) * lax.rsqrt(var + 1e-5) + be1[...]
    h = jnp.maximum(h, 0.0)
    h1_out[...] = h
    y2_out[...] = jnp.dot(h, w2l[...], preferred_element_type=jnp.float32)


def _tc2_body(s, c, h1, w2r, b2, g2, be2, out):
    cnt = jnp.maximum(c[:N, 0:1], 1.0)
    aggp = jnp.concatenate([s[0, :N], s[1, :N]], axis=1) / cnt
    h = (aggp
         + jnp.dot(h1[...], w2r[...], preferred_element_type=jnp.float32)
         + b2[...])
    mean = jnp.mean(h, axis=0, keepdims=True)
    var = jnp.mean((h - mean) ** 2, axis=0, keepdims=True)
    h = g2[...] * (h - mean) * lax.rsqrt(var + 1e-5) + be2[...]
    out[...] = jnp.maximum(h, 0.0)


_tc1 = pl.pallas_call(
    _tc1_body,
    out_shape=[jax.ShapeDtypeStruct((N, D1), jnp.float32),
               jax.ShapeDtypeStruct((N, D2), jnp.float32)],
)

_tc2 = pl.pallas_call(
    _tc2_body,
    out_shape=jax.ShapeDtypeStruct((N, D2), jnp.float32),
)


def kernel(x, edge_index, W1_l, W1_r, b1, g1, be1, W2_l, W2_r, b2, g2, be2):
    pad = EP - E
    src = jnp.concatenate(
        [edge_index[0].astype(jnp.int32), jnp.zeros((pad,), jnp.int32)]
    ).reshape(NS, NBLK, BC, K)
    dst = jnp.concatenate(
        [edge_index[1].astype(jnp.int32), jnp.full((pad,), N, jnp.int32)]
    ).reshape(NS, NBLK, BC, K)
    zf1 = jnp.zeros((NP, D1 // NC), jnp.float32)
    zc = jnp.zeros((NP, CW), jnp.float32)
    zf2 = jnp.zeros((NP, D2 // NC), jnp.float32)

    xh = jnp.stack([x[:, :D1 // 2], x[:, D1 // 2:]])
    sum1, cnt = _sc_agg_l1(src, dst, xh, zf1, zc)
    h1, y2 = _tc1(x, sum1, cnt,
                  W1_l, W1_r, b1.reshape(1, D1), g1.reshape(1, D1),
                  be1.reshape(1, D1), W2_l)
    yh = jnp.stack([y2[:, :D2 // 2], y2[:, D2 // 2:]])
    (sum2,) = _sc_agg_l2(src, dst, yh, zf2)
    out = _tc2(sum2, cnt, h1, W2_r, b2.reshape(1, D2),
               g2.reshape(1, D2), be2.reshape(1, D2))
    return out
